# Initial kernel scaffold; baseline (speedup 1.0000x reference)
import jax
import jax.numpy as jnp
from jax.experimental import pallas as pl


def kernel(input, rois, offset):
    raise NotImplementedError("write your pallas kernel here")


# speedup vs baseline: 5.9567x; 5.9567x over previous
import functools

import jax
import jax.numpy as jnp
import numpy as np
from jax import lax
from jax.experimental import pallas as pl
from jax.experimental.pallas import tpu as pltpu
from jax.experimental.pallas import tpu_sc as plsc

_SPATIAL_SCALE = 0.125
_P = 7
_S = 4
_TRANS_STD = 0.1
_PART_SIZE = 7

_NC = 2
_NS = 16
_NW = _NC * _NS
_L = 16

_SAMPLES = _S * _S
_BINS = _P * _P
_ROW_BLK = 128


def _weights_body(rois_ref, offx_ref, offy_ref, idx_ref, w00_ref, w01_ref,
                  w10_ref, w11_ref, *, H, W):
    f32 = jnp.float32
    cols = _BINS * _SAMPLES
    bi = rois_ref[:, 0:1].astype(jnp.int32)
    sw = jnp.round(rois_ref[:, 1:2]) * _SPATIAL_SCALE - 0.5
    sh = jnp.round(rois_ref[:, 2:3]) * _SPATIAL_SCALE - 0.5
    ew = (jnp.round(rois_ref[:, 3:4]) + 1.0) * _SPATIAL_SCALE - 0.5
    eh = (jnp.round(rois_ref[:, 4:5]) + 1.0) * _SPATIAL_SCALE - 0.5
    roi_w = jnp.maximum(ew - sw, 0.1)
    roi_h = jnp.maximum(eh - sh, 0.1)
    bin_w = roi_w / _P
    bin_h = roi_h / _P
    sub_w = bin_w / _S
    sub_h = bin_h / _S

    ci = lax.broadcasted_iota(jnp.int32, (_ROW_BLK, cols), 1)
    binc = ci // _SAMPLES
    s = ci - binc * _SAMPLES
    ih = s // _S
    iw = s - ih * _S
    phv = binc // _P
    pwv = binc - phv * _P
    phf = phv.astype(f32)
    pwf = pwv.astype(f32)
    ihf = ih.astype(f32)
    iwf = iw.astype(f32)

    tx = offx_ref[...] * _TRANS_STD
    ty = offy_ref[...] * _TRANS_STD
    wstart = pwf * bin_w + sw + tx * roi_w
    hstart = phf * bin_h + sh + ty * roi_h
    w = wstart + iwf * sub_w
    h = hstart + ihf * sub_h

    def _inrange(v, lim):
        return (v >= -0.5) & (v <= lim - 0.5)

    valid = (_inrange(w, W) & _inrange(h, H)).astype(f32)
    cw = sum(_inrange(wstart + float(j) * sub_w, W).astype(f32)
             for j in range(_S))
    ch = sum(_inrange(hstart + float(j) * sub_h, H).astype(f32)
             for j in range(_S))
    cnt = cw * ch

    wc = jnp.clip(w, 0.0, W - 1.0)
    hc = jnp.clip(h, 0.0, H - 1.0)
    x0 = jnp.floor(wc)
    y0 = jnp.floor(hc)
    dx = wc - x0
    dy = hc - y0
    idx_ref[...] = bi * (H * W) + y0.astype(jnp.int32) * W + x0.astype(jnp.int32)
    scale = valid / jnp.maximum(cnt, 1.0)
    w00_ref[...] = (1.0 - dx) * (1.0 - dy) * scale
    w01_ref[...] = dx * (1.0 - dy) * scale
    w10_ref[...] = (1.0 - dx) * dy * scale
    w11_ref[...] = dx * dy * scale


def _compute_weights(rois_p, offx, offy, H, W):
    NP = rois_p.shape[0]
    cols = _BINS * _SAMPLES
    grid = (NP // _ROW_BLK,)
    spec_r = pl.BlockSpec((_ROW_BLK, 5), lambda i: (i, 0))
    spec_c = pl.BlockSpec((_ROW_BLK, cols), lambda i: (i, 0))
    out_shapes = [jax.ShapeDtypeStruct((NP, cols), jnp.int32)] + \
                 [jax.ShapeDtypeStruct((NP, cols), jnp.float32)] * 4
    return pl.pallas_call(
        functools.partial(_weights_body, H=H, W=W),
        grid=grid,
        in_specs=[spec_r, spec_c, spec_c],
        out_specs=[spec_c] * 5,
        out_shape=out_shapes,
    )(rois_p, offx, offy)


def _sc_pool(tab, idx_flat, wts_flat, n_bins_padded, C):
    bpw = n_bins_padded // _NW
    chunk = 16
    nchunk = bpw // chunk
    ncc = C // _L
    mesh = plsc.VectorSubcoreMesh(core_axis_name="c", subcore_axis_name="s",
                                  num_cores=_NC, num_subcores=_NS)

    @functools.partial(
        pl.kernel,
        mesh=mesh,
        out_type=jax.ShapeDtypeStruct((n_bins_padded * C,), jnp.float32),
        scratch_types=[
            pltpu.VMEM((chunk * _SAMPLES,), jnp.int32),
            pltpu.VMEM((chunk * _SAMPLES * 4,), jnp.float32),
            pltpu.VMEM((_SAMPLES, 4 * C), jnp.float32),
            pltpu.VMEM((C,), jnp.float32),
            pltpu.SemaphoreType.DMA,
        ],
    )
    def body(tab_hbm, idx_hbm, wts_hbm, out_hbm, idx_v, wts_v, rows_v,
             out_v, sem):
        wid = lax.axis_index("s") * _NC + lax.axis_index("c")
        bin0 = wid * bpw

        @pl.loop(0, nchunk)
        def _chunk_loop(chi):
            cb = bin0 + chi * chunk
            pltpu.sync_copy(idx_hbm.at[pl.ds(cb * _SAMPLES, chunk * _SAMPLES)],
                            idx_v)
            pltpu.sync_copy(
                wts_hbm.at[pl.ds(cb * _SAMPLES * 4, chunk * _SAMPLES * 4)],
                wts_v)

            @pl.loop(0, chunk)
            def _bin_loop(b):
                pltpu.async_copy(
                    tab_hbm.at[idx_v.at[pl.ds(b * _SAMPLES, _SAMPLES)]],
                    rows_v, sem).wait()
                accs = [jnp.zeros((_L,), jnp.float32) for _ in range(ncc)]
                wvecs = [wts_v[pl.ds(b * (_SAMPLES * 4) + g * _L, _L)]
                         for g in range(_SAMPLES * 4 // _L)]
                for k in range(_SAMPLES):
                    for q in range(4):
                        j = k * 4 + q
                        wsp = lax.broadcast(wvecs[j // _L][j % _L], (_L,))
                        for c in range(ncc):
                            accs[c] = accs[c] + wsp * rows_v[
                                k, pl.ds(q * C + c * _L, _L)]
                for c in range(ncc):
                    out_v[pl.ds(c * _L, _L)] = accs[c]
                pltpu.sync_copy(out_v, out_hbm.at[pl.ds((cb + b) * C, C)])

    return body(tab, idx_flat, wts_flat)


def kernel(input, rois, offset):
    B, C, H, W = input.shape
    N = rois.shape[0]
    NP = ((N + _ROW_BLK - 1) // _ROW_BLK) * _ROW_BLK
    nbins = NP * _BINS
    V = B * H * W

    xt = jnp.transpose(input, (0, 2, 3, 1)).reshape(V, C)
    xtp = jnp.pad(xt, ((0, W + 2), (0, 0)))
    tab = jnp.concatenate(
        [xtp[0:V], xtp[1:V + 1], xtp[W:V + W], xtp[W + 1:V + W + 1]], axis=1)

    pr = np.arange(_P, dtype=np.float32)
    part = np.floor(pr / np.float32(_P) * np.float32(_PART_SIZE)).astype(np.int64)
    off_x = offset[:, 0][:, part, :][:, :, part].reshape(N, _BINS)
    off_y = offset[:, 1][:, part, :][:, :, part].reshape(N, _BINS)
    offx = jnp.repeat(off_x, _SAMPLES, axis=1)
    offy = jnp.repeat(off_y, _SAMPLES, axis=1)

    rois_p = jnp.pad(rois, ((0, NP - N), (0, 0)))
    offx = jnp.pad(offx, ((0, NP - N), (0, 0)))
    offy = jnp.pad(offy, ((0, NP - N), (0, 0)))

    idx, w00, w01, w10, w11 = _compute_weights(rois_p, offx, offy, H, W)

    idx_flat = idx.reshape(nbins * _SAMPLES)
    wts_flat = jnp.stack([w00, w01, w10, w11], axis=-1).reshape(
        nbins * _SAMPLES * 4)

    out_flat = _sc_pool(tab, idx_flat, wts_flat, nbins, C)

    out = out_flat.reshape(NP, _BINS, C)[:N]
    out = out.reshape(N, _P, _P, C)
    return jnp.transpose(out, (0, 3, 1, 2))


# speedup vs baseline: 6.8564x; 1.1510x over previous
import functools

import jax
import jax.numpy as jnp
import numpy as np
from jax import lax
from jax.experimental import pallas as pl
from jax.experimental.pallas import tpu as pltpu
from jax.experimental.pallas import tpu_sc as plsc

_SPATIAL_SCALE = 0.125
_P = 7
_S = 4
_TRANS_STD = 0.1
_PART_SIZE = 7

_NC = 2
_NS = 16
_NW = _NC * _NS
_L = 16

_SAMPLES = _S * _S
_BINS = _P * _P
_ROW_BLK = 128


def _weights_body(rois_ref, offx_ref, offy_ref, idx_ref, w00_ref, w01_ref,
                  w10_ref, w11_ref, *, H, W):
    f32 = jnp.float32
    cols = _BINS * _SAMPLES
    bi = rois_ref[:, 0:1].astype(jnp.int32)
    sw = jnp.round(rois_ref[:, 1:2]) * _SPATIAL_SCALE - 0.5
    sh = jnp.round(rois_ref[:, 2:3]) * _SPATIAL_SCALE - 0.5
    ew = (jnp.round(rois_ref[:, 3:4]) + 1.0) * _SPATIAL_SCALE - 0.5
    eh = (jnp.round(rois_ref[:, 4:5]) + 1.0) * _SPATIAL_SCALE - 0.5
    roi_w = jnp.maximum(ew - sw, 0.1)
    roi_h = jnp.maximum(eh - sh, 0.1)
    bin_w = roi_w / _P
    bin_h = roi_h / _P
    sub_w = bin_w / _S
    sub_h = bin_h / _S

    ci = lax.broadcasted_iota(jnp.int32, (_ROW_BLK, cols), 1)
    binc = ci // _SAMPLES
    s = ci - binc * _SAMPLES
    ih = s // _S
    iw = s - ih * _S
    phv = binc // _P
    pwv = binc - phv * _P
    phf = phv.astype(f32)
    pwf = pwv.astype(f32)
    ihf = ih.astype(f32)
    iwf = iw.astype(f32)

    tx = offx_ref[...] * _TRANS_STD
    ty = offy_ref[...] * _TRANS_STD
    wstart = pwf * bin_w + sw + tx * roi_w
    hstart = phf * bin_h + sh + ty * roi_h
    w = wstart + iwf * sub_w
    h = hstart + ihf * sub_h

    def _inrange(v, lim):
        return (v >= -0.5) & (v <= lim - 0.5)

    valid = (_inrange(w, W) & _inrange(h, H)).astype(f32)
    cw = sum(_inrange(wstart + float(j) * sub_w, W).astype(f32)
             for j in range(_S))
    ch = sum(_inrange(hstart + float(j) * sub_h, H).astype(f32)
             for j in range(_S))
    cnt = cw * ch

    wc = jnp.clip(w, 0.0, W - 1.0)
    hc = jnp.clip(h, 0.0, H - 1.0)
    x0 = jnp.floor(wc)
    y0 = jnp.floor(hc)
    dx = wc - x0
    dy = hc - y0
    idx_ref[...] = bi * (H * W) + y0.astype(jnp.int32) * W + x0.astype(jnp.int32)
    scale = valid / jnp.maximum(cnt, 1.0)
    w00_ref[...] = (1.0 - dx) * (1.0 - dy) * scale
    w01_ref[...] = dx * (1.0 - dy) * scale
    w10_ref[...] = (1.0 - dx) * dy * scale
    w11_ref[...] = dx * dy * scale


def _compute_weights(rois_p, offx, offy, H, W):
    NP = rois_p.shape[0]
    cols = _BINS * _SAMPLES
    grid = (NP // _ROW_BLK,)
    spec_r = pl.BlockSpec((_ROW_BLK, 5), lambda i: (i, 0))
    spec_c = pl.BlockSpec((_ROW_BLK, cols), lambda i: (i, 0))
    out_shapes = [jax.ShapeDtypeStruct((NP, cols), jnp.int32)] + \
                 [jax.ShapeDtypeStruct((NP, cols), jnp.float32)] * 4
    return pl.pallas_call(
        functools.partial(_weights_body, H=H, W=W),
        grid=grid,
        in_specs=[spec_r, spec_c, spec_c],
        out_specs=[spec_c] * 5,
        out_shape=out_shapes,
    )(rois_p, offx, offy)


def _sc_pool(tab, idx_flat, wts_flat, n_bins_padded, C):
    bpw = n_bins_padded // _NW
    chunk = 16
    nchunk = bpw // chunk
    ncc = C // _L
    mesh = plsc.VectorSubcoreMesh(core_axis_name="c", subcore_axis_name="s",
                                  num_cores=_NC, num_subcores=_NS)

    @functools.partial(
        pl.kernel,
        mesh=mesh,
        out_type=jax.ShapeDtypeStruct((n_bins_padded * C,), jnp.float32),
        scratch_types=[
            pltpu.VMEM((chunk * _SAMPLES,), jnp.int32),
            pltpu.VMEM((chunk * _SAMPLES * 4,), jnp.float32),
            pltpu.VMEM((2, _SAMPLES, 4 * C), jnp.float32),
            pltpu.VMEM((2, C), jnp.float32),
            pltpu.SemaphoreType.DMA,
            pltpu.SemaphoreType.DMA,
            pltpu.SemaphoreType.DMA,
            pltpu.SemaphoreType.DMA,
        ],
    )
    def body(tab_hbm, idx_hbm, wts_hbm, out_hbm, idx_v, wts_v, rows_v,
             out_v, gsem0, gsem1, osem0, osem1):
        wid = lax.axis_index("s") * _NC + lax.axis_index("c")
        bin0 = wid * bpw
        gsems = (gsem0, gsem1)
        osems = (osem0, osem1)

        def gather(b, buf, sem):
            return pltpu.async_copy(
                tab_hbm.at[idx_v.at[pl.ds(b * _SAMPLES, _SAMPLES)]],
                rows_v.at[buf], sem)

        def accumulate(cb, b, buf, seen_prior):
            accs = [jnp.zeros((_L,), jnp.float32) for _ in range(ncc)]
            wvecs = [wts_v[pl.ds(b * (_SAMPLES * 4) + g * _L, _L)]
                     for g in range(_SAMPLES * 4 // _L)]
            for k in range(_SAMPLES):
                for q in range(4):
                    j = k * 4 + q
                    wsp = lax.broadcast(wvecs[j // _L][j % _L], (_L,))
                    for c in range(ncc):
                        accs[c] = accs[c] + wsp * rows_v[
                            buf, k, pl.ds(q * C + c * _L, _L)]
            @pl.when(seen_prior)
            def _():
                pltpu.make_async_copy(
                    out_v.at[buf], out_hbm.at[pl.ds(0, C)],
                    osems[buf]).wait()
            for c in range(ncc):
                out_v[buf, pl.ds(c * _L, _L)] = accs[c]
            pltpu.async_copy(out_v.at[buf],
                             out_hbm.at[pl.ds((cb + b) * C, C)], osems[buf])

        @pl.loop(0, nchunk)
        def _chunk_loop(chi):
            cb = bin0 + chi * chunk
            pltpu.sync_copy(idx_hbm.at[pl.ds(cb * _SAMPLES, chunk * _SAMPLES)],
                            idx_v)
            pltpu.sync_copy(
                wts_hbm.at[pl.ds(cb * _SAMPLES * 4, chunk * _SAMPLES * 4)],
                wts_v)
            gather(0, 0, gsems[0])

            @pl.loop(0, chunk // 2)
            def _pair_loop(p):
                seen = jnp.logical_or(chi > 0, p > 0)
                gather(2 * p + 1, 1, gsems[1])
                pltpu.make_async_copy(
                    tab_hbm.at[idx_v.at[pl.ds(0, _SAMPLES)]],
                    rows_v.at[0], gsems[0]).wait()
                accumulate(cb, 2 * p, 0, seen)

                @pl.when(p < chunk // 2 - 1)
                def _():
                    gather(2 * p + 2, 0, gsems[0])
                pltpu.make_async_copy(
                    tab_hbm.at[idx_v.at[pl.ds(0, _SAMPLES)]],
                    rows_v.at[1], gsems[1]).wait()
                accumulate(cb, 2 * p + 1, 1, seen)

        for buf in range(2):
            pltpu.make_async_copy(out_v.at[buf], out_hbm.at[pl.ds(0, C)],
                                  osems[buf]).wait()

    return body(tab, idx_flat, wts_flat)


def kernel(input, rois, offset):
    B, C, H, W = input.shape
    N = rois.shape[0]
    NP = ((N + _ROW_BLK - 1) // _ROW_BLK) * _ROW_BLK
    nbins = NP * _BINS
    V = B * H * W

    xt = jnp.transpose(input, (0, 2, 3, 1)).reshape(V, C)
    xtp = jnp.pad(xt, ((0, W + 2), (0, 0)))
    tab = jnp.concatenate(
        [xtp[0:V], xtp[1:V + 1], xtp[W:V + W], xtp[W + 1:V + W + 1]], axis=1)

    pr = np.arange(_P, dtype=np.float32)
    part = np.floor(pr / np.float32(_P) * np.float32(_PART_SIZE)).astype(np.int64)
    off_x = offset[:, 0][:, part, :][:, :, part].reshape(N, _BINS)
    off_y = offset[:, 1][:, part, :][:, :, part].reshape(N, _BINS)
    offx = jnp.repeat(off_x, _SAMPLES, axis=1)
    offy = jnp.repeat(off_y, _SAMPLES, axis=1)

    rois_p = jnp.pad(rois, ((0, NP - N), (0, 0)))
    offx = jnp.pad(offx, ((0, NP - N), (0, 0)))
    offy = jnp.pad(offy, ((0, NP - N), (0, 0)))

    idx, w00, w01, w10, w11 = _compute_weights(rois_p, offx, offy, H, W)

    idx_flat = idx.reshape(nbins * _SAMPLES)
    wts_flat = jnp.stack([w00, w01, w10, w11], axis=-1).reshape(
        nbins * _SAMPLES * 4)

    out_flat = _sc_pool(tab, idx_flat, wts_flat, nbins, C)

    out = out_flat.reshape(NP, _BINS, C)[:N]
    out = out.reshape(N, _P, _P, C)
    return jnp.transpose(out, (0, 3, 1, 2))
